# Initial kernel scaffold; baseline (speedup 1.0000x reference)
import jax
import jax.numpy as jnp
from jax.experimental import pallas as pl


def kernel(input, mipmap_0, mipmap_1, mipmap_2, mipmap_3):
    raise NotImplementedError("write your pallas kernel here")


# speedup vs baseline: 73.6105x; 73.6105x over previous
import functools

import jax
import jax.numpy as jnp
from jax import lax
from jax.experimental import pallas as pl
from jax.experimental.pallas import tpu as pltpu
from jax.experimental.pallas import tpu_sc as plsc

SIZE = 1024
DEPTH = 16
NPIX = 4 * 512 * 512
LEVEL_W = [SIZE >> l for l in range(4)]
LEVEL_BASE = [0, 1048576, 1310720, 1376256]
TABLE_ROWS = 1392640

P = 256
G = P // 16
NK = 16
CPB = 262144
CH_STRIDE = 262144
B_STRIDE = 16 * 262144


def _sc_body(u_hbm, v_hbm, table_hbm, out_hbm,
             u_v, v_v, idx_v, w_v, rows_v, out_v, gsem):
    info = plsc.get_sparse_core_info()
    nc = info.num_cores
    wid = lax.axis_index("s") * nc + lax.axis_index("c")
    npix_w = NPIX // (nc * info.num_subcores)
    nblk = npix_w // P

    def block_body(blk, _):
        pix0 = wid * npix_w + blk * P
        pltpu.sync_copy(u_hbm.at[pl.ds(pix0, P)], u_v)
        pltpu.sync_copy(v_hbm.at[pl.ds(pix0, P)], v_v)

        iota = lax.iota(jnp.int32, 16)

        def idx_body(g, _):
            p0 = g * 16
            ux = u_v[pl.ds(p0, 16)]
            uy = v_v[pl.ds(p0, 16)]
            gx = 2.0 * ux - 1.0
            gy = 2.0 * uy - 1.0
            for l in range(4):
                w = LEVEL_W[l]
                base = LEVEL_BASE[l]
                ix = ((gx + 1.0) * w - 1.0) / 2.0
                iy = ((gy + 1.0) * w - 1.0) / 2.0
                tx = ix.astype(jnp.int32)
                x0 = tx - jnp.where(tx.astype(jnp.float32) > ix, 1, 0)
                ty = iy.astype(jnp.int32)
                y0 = ty - jnp.where(ty.astype(jnp.float32) > iy, 1, 0)
                wx1 = ix - x0.astype(jnp.float32)
                wx0 = 1.0 - wx1
                wy1 = iy - y0.astype(jnp.float32)
                wy0 = 1.0 - wy1
                ax0 = wx0 * jnp.where(x0 >= 0, 1.0, 0.0)
                ax1 = wx1 * jnp.where(x0 <= w - 2, 1.0, 0.0)
                ay0 = wy0 * jnp.where(y0 >= 0, 1.0, 0.0)
                ay1 = wy1 * jnp.where(y0 <= w - 2, 1.0, 0.0)
                xc0 = jnp.maximum(x0, 0)
                xc1 = jnp.minimum(x0 + 1, w - 1)
                yc0 = jnp.maximum(y0, 0)
                yc1 = jnp.minimum(y0 + 1, w - 1)
                r0 = yc0 * w + (base + xc0)
                r1 = yc0 * w + (base + xc1)
                r2 = yc1 * w + (base + xc0)
                r3 = yc1 * w + (base + xc1)
                for ci, (rr, ww) in enumerate(
                        ((r0, ax0 * ay0), (r1, ax1 * ay0),
                         (r2, ax0 * ay1), (r3, ax1 * ay1))):
                    k = l * 4 + ci
                    plsc.store_scatter(idx_v, [(p0 + iota) * NK + k], rr)
                    plsc.store_scatter(w_v, [(p0 + iota) * NK + k], ww)
            return 0

        lax.fori_loop(0, G, idx_body, 0, unroll=False)

        copies = []
        for j in range(NK * P // 128):
            copies.append(pltpu.async_copy(
                table_hbm.at[idx_v.at[pl.ds(j * 128, 128)]],
                rows_v.at[pl.ds(j * 128, 128)], gsem))
        for cp in copies:
            cp.wait()

        def acc_body(p, _):
            base = p * NK
            acc = jnp.zeros((16,), jnp.float32)
            for k in range(NK):
                r = rows_v[base + k, :]
                wk = plsc.load_gather(w_v, [jnp.broadcast_to(base + k, (16,))])
                acc = acc + wk * r
            plsc.store_scatter(out_v, [iota * P + p], acc)
            return 0

        lax.fori_loop(0, P, acc_body, 0, unroll=False)

        b = pix0 // CPB
        pib = pix0 % CPB
        obase = b * B_STRIDE + pib
        for c in range(16):
            pltpu.sync_copy(out_v.at[pl.ds(c * P, P)],
                            out_hbm.at[pl.ds(obase + c * CH_STRIDE, P)])
        return 0

    lax.fori_loop(0, nblk, block_body, 0, unroll=False)


@jax.jit
def _run(u, v, table):
    mesh = plsc.VectorSubcoreMesh(core_axis_name="c", subcore_axis_name="s")
    f = pl.kernel(
        _sc_body,
        out_type=jax.ShapeDtypeStruct((NPIX * 16,), jnp.float32),
        mesh=mesh,
        compiler_params=pltpu.CompilerParams(
            needs_layout_passes=False, use_tc_tiling_on_sc=False),
        scratch_types=[
            pltpu.VMEM((P,), jnp.float32),
            pltpu.VMEM((P,), jnp.float32),
            pltpu.VMEM((NK * P,), jnp.int32),
            pltpu.VMEM((NK * P,), jnp.float32),
            pltpu.VMEM((NK * P, 16), jnp.float32),
            pltpu.VMEM((16 * P,), jnp.float32),
            pltpu.SemaphoreType.DMA,
        ],
    )
    return f(u, v, table)


def kernel(input, mipmap_0, mipmap_1, mipmap_2, mipmap_3):
    u = input[..., 0].reshape(-1)
    v = input[..., 1].reshape(-1)
    table = jnp.concatenate([
        m[0].transpose(1, 2, 0).reshape(-1, DEPTH)
        for m in (mipmap_0, mipmap_1, mipmap_2, mipmap_3)
    ], axis=0)
    out = _run(u, v, table)
    return out.reshape(4, DEPTH, 512, 512)
